# Initial kernel scaffold; baseline (speedup 1.0000x reference)
import jax
import jax.numpy as jnp
from jax.experimental import pallas as pl


def kernel(x, emb, W, b):
    raise NotImplementedError("write your pallas kernel here")


# speedup vs baseline: 13.4521x; 13.4521x over previous
import functools

import jax
import jax.numpy as jnp
from jax import lax
from jax.experimental import pallas as pl
from jax.experimental.pallas import tpu as pltpu
from jax.experimental.pallas import tpu_sc as plsc

VOCAB = 100000
D = 128
B = 4096
SEQ = 200
LANES = 16
NCHUNK = D // LANES

_info = plsc.get_sparse_core_info()
NC = _info.num_cores
NS = _info.num_subcores
NW = NC * NS
BPW = B // NW

_mesh = plsc.VectorSubcoreMesh(core_axis_name="c", subcore_axis_name="s")

SEQ_A = 128
SEQ_B = SEQ - SEQ_A


def _sc_pool_body(x_hbm, emb_hbm, out_hbm, idx_v, rows0, rows1, acc_v,
                  sem0, sem1):
    wid = lax.axis_index("s") * NC + lax.axis_index("c")
    base = wid * BPW

    pltpu.sync_copy(x_hbm.at[pl.ds(base, BPW)], idx_v)

    def fire(local, buf, sem):
        pltpu.async_copy(emb_hbm.at[idx_v.at[local, pl.ds(0, SEQ_A)]],
                         buf.at[pl.ds(0, SEQ_A)], sem)
        pltpu.async_copy(emb_hbm.at[idx_v.at[local, pl.ds(SEQ_A, SEQ_B)]],
                         buf.at[pl.ds(SEQ_A, SEQ_B)], sem)

    def drain(buf, sem):
        pltpu.make_async_copy(emb_hbm.at[pl.ds(0, SEQ)], buf, sem).wait()

    def reduce_into(local, buf):
        def body(l, accs):
            return tuple(accs[j] + buf[l, pl.ds(LANES * j, LANES)]
                         for j in range(NCHUNK))
        accs = lax.fori_loop(
            0, SEQ, body,
            tuple(jnp.zeros((LANES,), jnp.float32) for _ in range(NCHUNK)),
            unroll=2)
        for j in range(NCHUNK):
            acc_v[local, pl.ds(LANES * j, LANES)] = accs[j]

    fire(0, rows0, sem0)

    def outer(k, carry):
        i = 2 * k
        fire(i + 1, rows1, sem1)
        drain(rows0, sem0)
        reduce_into(i, rows0)

        @pl.when(i + 2 < BPW)
        def _():
            fire(i + 2, rows0, sem0)

        drain(rows1, sem1)
        reduce_into(i + 1, rows1)
        return carry

    lax.fori_loop(0, BPW // 2, outer, 0)
    pltpu.sync_copy(acc_v, out_hbm.at[pl.ds(base, BPW)])


_sc_pool = functools.partial(
    pl.kernel,
    out_type=jax.ShapeDtypeStruct((B, D), jnp.float32),
    mesh=_mesh,
    scratch_types=[
        pltpu.VMEM((BPW, SEQ), jnp.int32),
        pltpu.VMEM((SEQ, D), jnp.float32),
        pltpu.VMEM((SEQ, D), jnp.float32),
        pltpu.VMEM((BPW, D), jnp.float32),
        pltpu.SemaphoreType.DMA,
        pltpu.SemaphoreType.DMA,
    ],
)(_sc_pool_body)


def _fc_body(p_ref, w_ref, b_ref, o_ref):
    o_ref[...] = jnp.dot(p_ref[...], w_ref[...],
                         preferred_element_type=jnp.float32) + b_ref[...]


def _fc(p, w, bias2d):
    grid = 8
    return pl.pallas_call(
        _fc_body,
        grid=(grid,),
        in_specs=[
            pl.BlockSpec((B // grid, D), lambda i: (i, 0)),
            pl.BlockSpec((D, D), lambda i: (0, 0)),
            pl.BlockSpec((1, D), lambda i: (0, 0)),
        ],
        out_specs=pl.BlockSpec((B // grid, D), lambda i: (i, 0)),
        out_shape=jax.ShapeDtypeStruct((B, D), jnp.float32),
    )(p, w, bias2d)


def kernel(x, emb, W, b):
    x = x.astype(jnp.int32)
    p_sum = _sc_pool(x, emb)
    return _fc(p_sum, W * jnp.float32(1.0 / SEQ), b.reshape(1, D))
